# Initial kernel scaffold; baseline (speedup 1.0000x reference)
import jax
import jax.numpy as jnp
from jax.experimental import pallas as pl


def kernel(sensory_input, W1, b1, W2, b2, Wq, bq, Wk, bk, Wv, bv, Wo, bo, W3, b3, W4, b4):
    raise NotImplementedError("write your pallas kernel here")


# speedup vs baseline: 1.8448x; 1.8448x over previous
import jax
import jax.numpy as jnp
import numpy as np
from jax.experimental import pallas as pl

B = 1024
D = 512
H = 8
DH = D // H


def _mm_t(a, w):
    return jax.lax.dot_general(a, w, (((1,), (1,)), ((), ())),
                               preferred_element_type=jnp.float32)


def _fused_kernel(x_ref, W1_ref, b1_ref, W2_ref, b2_ref,
                  Wq_ref, bq_ref, Wk_ref, bk_ref, Wv_ref, bv_ref,
                  Wo_ref, bo_ref, W3p_ref, W3a_ref, b3_ref,
                  W4_ref, b4_ref, out_ref):
    x = x_ref[...]
    h = jnp.maximum(_mm_t(x, W1_ref[...]) + b1_ref[...], 0.0)
    patterns = jnp.maximum(_mm_t(h, W2_ref[...]) + b2_ref[...], 0.0)

    q = _mm_t(x, Wq_ref[...]) + bq_ref[...]
    k = _mm_t(x, Wk_ref[...]) + bk_ref[...]
    v = _mm_t(x, Wv_ref[...]) + bv_ref[...]

    scale = np.float32(1.0 / np.sqrt(DH))
    head_outs = []
    for hh in range(H):
        qh = q[:, hh * DH:(hh + 1) * DH] * scale
        kh = k[:, hh * DH:(hh + 1) * DH]
        vh = v[:, hh * DH:(hh + 1) * DH]
        s = jax.lax.dot_general(qh, kh, (((1,), (1,)), ((), ())),
                                preferred_element_type=jnp.float32)
        m = jnp.max(s, axis=-1, keepdims=True)
        e = jnp.exp(s - m)
        p = e / jnp.sum(e, axis=-1, keepdims=True)
        head_outs.append(jnp.dot(p, vh, preferred_element_type=jnp.float32))
    att = jnp.concatenate(head_outs, axis=-1)
    attended = _mm_t(att, Wo_ref[...]) + bo_ref[...]

    h2 = jnp.maximum(_mm_t(patterns, W3p_ref[...])
                     + _mm_t(attended, W3a_ref[...]) + b3_ref[...], 0.0)
    out_ref[...] = jnp.tanh(_mm_t(h2, W4_ref[...]) + b4_ref[...])


def kernel(sensory_input, W1, b1, W2, b2, Wq, bq, Wk, bk, Wv, bv, Wo, bo, W3, b3, W4, b4):
    W3p = W3[:, :128]
    W3a = W3[:, 128:]
    args = (sensory_input, W1, b1.reshape(1, -1), W2, b2.reshape(1, -1),
            Wq, bq.reshape(1, -1), Wk, bk.reshape(1, -1), Wv, bv.reshape(1, -1),
            Wo, bo.reshape(1, -1), W3p, W3a, b3.reshape(1, -1),
            W4, b4.reshape(1, -1))
    return pl.pallas_call(
        _fused_kernel,
        out_shape=jax.ShapeDtypeStruct((B, D), jnp.float32),
    )(*args)


# speedup vs baseline: 2.0226x; 1.0964x over previous
import jax
import jax.numpy as jnp
import numpy as np
from jax.experimental import pallas as pl

B = 1024
D = 512
H = 8
DH = D // H


def _mm_t(a, w):
    return jax.lax.dot_general(a, w, (((1,), (1,)), ((), ())),
                               preferred_element_type=jnp.float32)


def _fused_kernel(x_ref, W1_ref, b1_ref, W2_ref, b2_ref,
                  Wq_ref, bq_ref, Wk_ref, bk_ref, Wv_ref, bv_ref,
                  Wo_ref, bo_ref, W3p_ref, W3a_ref, b3_ref,
                  W4_ref, b4_ref, out_ref):
    x = x_ref[...]
    h = jnp.maximum(_mm_t(x, W1_ref[...]) + b1_ref[...], 0.0)
    patterns = jnp.maximum(_mm_t(h, W2_ref[...]) + b2_ref[...], 0.0)

    q = _mm_t(x, Wq_ref[...]) + bq_ref[...]
    k = _mm_t(x, Wk_ref[...]) + bk_ref[...]
    v = _mm_t(x, Wv_ref[...]) + bv_ref[...]

    scale = np.float32(1.0 / np.sqrt(DH))
    head_outs = []
    for hh in range(H):
        qh = q[:, hh * DH:(hh + 1) * DH] * scale
        kh = k[:, hh * DH:(hh + 1) * DH]
        vh = v[:, hh * DH:(hh + 1) * DH]
        s = jax.lax.dot_general(qh, kh, (((1,), (1,)), ((), ())),
                                preferred_element_type=jnp.float32)
        m = jnp.max(s, axis=-1, keepdims=True)
        e = jnp.exp(s - m)
        r = 1.0 / jnp.sum(e, axis=-1, keepdims=True)
        o = jnp.dot(e, vh, preferred_element_type=jnp.float32)
        head_outs.append(o * r)
    att = jnp.concatenate(head_outs, axis=-1)
    attended = _mm_t(att, Wo_ref[...]) + bo_ref[...]

    h2 = jnp.maximum(_mm_t(patterns, W3p_ref[...])
                     + _mm_t(attended, W3a_ref[...]) + b3_ref[...], 0.0)
    out_ref[...] = jnp.tanh(_mm_t(h2, W4_ref[...]) + b4_ref[...])


def kernel(sensory_input, W1, b1, W2, b2, Wq, bq, Wk, bk, Wv, bv, Wo, bo, W3, b3, W4, b4):
    W3p = W3[:, :128]
    W3a = W3[:, 128:]
    args = (sensory_input, W1, b1.reshape(1, -1), W2, b2.reshape(1, -1),
            Wq, bq.reshape(1, -1), Wk, bk.reshape(1, -1), Wv, bv.reshape(1, -1),
            Wo, bo.reshape(1, -1), W3p, W3a, b3.reshape(1, -1),
            W4, b4.reshape(1, -1))
    return pl.pallas_call(
        _fused_kernel,
        out_shape=jax.ShapeDtypeStruct((B, D), jnp.float32),
    )(*args)
